# baseline (speedup 1.0000x reference)
import functools

import jax
import jax.numpy as jnp
from jax import lax
from jax.experimental import pallas as pl
from jax.experimental.pallas import tpu as pltpu
from jax.experimental.pallas import tpu_sc as plsc

_VOCAB = 1000000
_EMBED_DIM = 64
_BATCH = 4096
_HIST = 200
_B = _BATCH * _HIST

_NC = 2
_NS = 16
_NW = _NC * _NS
_BPW = _BATCH // _NW
_PER_W = _BPW * _HIST
_G = _HIST
_NG = _BPW
_PAIR_ROWS = _VOCAB // 2
_NBLK = _G // 16


def _build():
    mesh = plsc.VectorSubcoreMesh(core_axis_name="c", subcore_axis_name="s")

    @functools.partial(
        pl.kernel,
        out_type=jax.ShapeDtypeStruct((_BATCH, _HIST, _EMBED_DIM),
                                      jnp.float32),
        mesh=mesh,
        scratch_types=[
            pltpu.VMEM((208,), jnp.int32),
            pltpu.VMEM((208,), jnp.int32),
            pltpu.VMEM((208,), jnp.int32),
            pltpu.VMEM((208,), jnp.int32),
            pltpu.VMEM((_G, 128), jnp.float32),
            pltpu.VMEM((_G, 128), jnp.float32),
            pltpu.VMEM((_G, _EMBED_DIM), jnp.float32),
            pltpu.VMEM((_G, _EMBED_DIM), jnp.float32),
            pltpu.SemaphoreType.DMA,
            pltpu.SemaphoreType.DMA,
            pltpu.SemaphoreType.DMA,
            pltpu.SemaphoreType.DMA,
            pltpu.SemaphoreType.DMA,
            pltpu.SemaphoreType.DMA,
        ],
        compiler_params=pltpu.CompilerParams(needs_layout_passes=False),
    )
    def gather_kernel(table_hbm, idx_hbm, out_hbm,
                      ix0, ix1, pr0, pr1, buf0, buf1, stg0, stg1,
                      g0, g1, s0, s1, x0, x1):
        wid = lax.axis_index("s") * _NC + lax.axis_index("c")
        base = wid * _PER_W
        bbase = wid * _BPW
        iota = lax.iota(jnp.int32, 16)
        tail_mask = iota < 8

        def load_ix(gi, ix):
            pltpu.sync_copy(idx_hbm.at[pl.ds(base + gi * _G, _G)],
                            ix.at[pl.ds(0, _G)])

        def start_ix(gi, ix, xsem):
            pltpu.async_copy(idx_hbm.at[pl.ds(base + gi * _G, _G)],
                             ix.at[pl.ds(0, _G)], xsem)

        def wait_ix(gi, ix, xsem):
            pltpu.make_async_copy(idx_hbm.at[pl.ds(base + gi * _G, _G)],
                                  ix.at[pl.ds(0, _G)], xsem).wait()

        def derive(ix, pr):
            def blk(i, carry):
                pr[pl.ds(i * 16, 16)] = jax.lax.shift_right_logical(
                    ix[pl.ds(i * 16, 16)], 1)
                return carry
            lax.fori_loop(0, 13, blk, 0)

        def fire_gathers(pr, buf, gsem):
            pltpu.async_copy(
                table_hbm.at[pr.at[pl.ds(0, 128)]],
                buf.at[pl.ds(0, 128)], gsem)
            pltpu.async_copy(
                table_hbm.at[pr.at[pl.ds(128, _G - 128)]],
                buf.at[pl.ds(128, _G - 128)], gsem)

        def drain_gathers(buf, gsem):
            pltpu.make_async_copy(
                table_hbm.at[pl.ds(0, 128)],
                buf.at[pl.ds(0, 128)], gsem).wait()
            pltpu.make_async_copy(
                table_hbm.at[pl.ds(0, _G - 128)],
                buf.at[pl.ds(128, _G - 128)], gsem).wait()

        def fix_block(buf, stg, rows, src0, mask):
            def quad(q, cols):
                src, dst = cols
                for _ in range(16):
                    v = plsc.load_gather(buf, [rows, src], mask=mask)
                    plsc.store_scatter(stg, [rows, dst], v, mask=mask)
                    src = src + 1
                    dst = dst + 1
                return (src, dst)
            lax.fori_loop(0, 4, quad, (src0, iota * 0))

        def fix_group(ix, buf, stg):
            def blk(r0, carry):
                rows = r0 * 16 + iota
                halves = jnp.bitwise_and(ix[pl.ds(r0 * 16, 16)], 1)
                fix_block(buf, stg, rows, halves << 6, None)
                return carry
            lax.fori_loop(0, _NBLK, blk, 0)
            rows = _NBLK * 16 + iota
            halves = jnp.bitwise_and(ix[pl.ds(_NBLK * 16, 16)], 1)
            fix_block(buf, stg, rows, halves << 6, tail_mask)

        def start_store(gi, stg, ssem):
            pltpu.async_copy(stg, out_hbm.at[bbase + gi], ssem)

        def drain_store(gi, stg, ssem):
            pltpu.make_async_copy(stg, out_hbm.at[bbase + gi], ssem).wait()

        load_ix(0, ix0)
        derive(ix0, pr0)
        fire_gathers(pr0, buf0, g0)
        start_ix(1, ix1, x1)

        def visit(gi, ix_c, pr_c, buf_c, stg_c, gsem_c, ssem_c, xsem_c,
                  ix_n, pr_n, buf_n, gsem_n, xsem_n, k):
            @pl.when(gi < _NG - 1)
            def _():
                wait_ix(gi + 1, ix_n, xsem_n)
                derive(ix_n, pr_n)

            drain_gathers(buf_c, gsem_c)

            @pl.when(gi < _NG - 1)
            def _():
                fire_gathers(pr_n, buf_n, gsem_n)

            @pl.when(gi >= 2)
            def _():
                drain_store(gi - 2, stg_c, ssem_c)

            fix_group(ix_c, buf_c, stg_c)

            @pl.when(gi < _NG - 2)
            def _():
                start_ix(gi + 2, ix_c, xsem_c)

            start_store(gi, stg_c, ssem_c)

        def body_k(k, carry):
            a = 2 * k
            visit(a, ix0, pr0, buf0, stg0, g0, s0, x0,
                  ix1, pr1, buf1, g1, x1, k)
            visit(a + 1, ix1, pr1, buf1, stg1, g1, s1, x1,
                  ix0, pr0, buf0, g0, x0, k)
            return carry

        lax.fori_loop(0, _NG // 2, body_k, 0)
        drain_store(_NG - 2, stg0, s0)
        drain_store(_NG - 1, stg1, s1)

    return gather_kernel


_gather = _build()


def kernel(inputs, W):
    idx = inputs.reshape(-1).astype(jnp.int32)
    table = W.reshape(_PAIR_ROWS, 128)
    return _gather(table, idx)


# speedup vs baseline: 2.2584x; 2.2584x over previous
import functools

import jax
import jax.numpy as jnp
from jax import lax
from jax.experimental import pallas as pl
from jax.experimental.pallas import tpu as pltpu
from jax.experimental.pallas import tpu_sc as plsc

_VOCAB = 1000000
_EMBED_DIM = 64
_BATCH = 4096
_HIST = 200
_B = _BATCH * _HIST

_NC = 2
_NS = 16
_NW = _NC * _NS
_BPW = _BATCH // _NW
_PER_W = _BPW * _HIST
_NB = 2
_GROWS = _NB * _HIST
_CHUNK = 80
_GCH = _GROWS // _CHUNK
_NG = _BPW // _NB


def _build():
    mesh = plsc.VectorSubcoreMesh(core_axis_name="c", subcore_axis_name="s")

    @functools.partial(
        pl.kernel,
        out_type=jax.ShapeDtypeStruct((_BATCH, _HIST, _EMBED_DIM),
                                      jnp.float32),
        mesh=mesh,
        scratch_types=[
            pltpu.VMEM((_PER_W,), jnp.int32),
            pltpu.VMEM((_GROWS, _EMBED_DIM), jnp.float32),
            pltpu.VMEM((_GROWS, _EMBED_DIM), jnp.float32),
            pltpu.SemaphoreType.DMA,
            pltpu.SemaphoreType.DMA,
            pltpu.SemaphoreType.DMA,
            pltpu.SemaphoreType.DMA,
        ],
        compiler_params=pltpu.CompilerParams(use_tc_tiling_on_sc=False),
    )
    def gather_kernel(table_hbm, idx_hbm, out_hbm, idx_v, buf0, buf1,
                      g0, g1, s0, s1):
        wid = lax.axis_index("s") * _NC + lax.axis_index("c")
        base = wid * _PER_W
        bbase = wid * _BPW
        pltpu.sync_copy(idx_hbm.at[pl.ds(base, _PER_W)], idx_v)

        def fire_gathers(gi, buf, gsem):
            for j in range(_GCH):
                off = gi * _GROWS + j * _CHUNK
                pltpu.async_copy(
                    table_hbm.at[idx_v.at[pl.ds(off, _CHUNK)]],
                    buf.at[pl.ds(j * _CHUNK, _CHUNK)],
                    gsem)

        def drain_gathers(buf, gsem):
            for t in range(_NB):
                pltpu.make_async_copy(
                    out_hbm.at[bbase],
                    buf.at[pl.ds(t * _HIST, _HIST)], gsem).wait()

        def start_store(gi, buf, ssem):
            for t in range(_NB):
                pltpu.async_copy(
                    buf.at[pl.ds(t * _HIST, _HIST)],
                    out_hbm.at[bbase + gi * _NB + t], ssem)

        def drain_store(gi, buf, ssem):
            for t in range(_NB):
                pltpu.make_async_copy(
                    buf.at[pl.ds(t * _HIST, _HIST)],
                    out_hbm.at[bbase + gi * _NB + t], ssem).wait()

        fire_gathers(0, buf0, g0)

        def body_k(k, carry):
            a = 2 * k
            b = a + 1
            drain_gathers(buf0, g0)
            start_store(a, buf0, s0)

            @pl.when(k > 0)
            def _():
                drain_store(a - 1, buf1, s1)

            fire_gathers(b, buf1, g1)
            drain_gathers(buf1, g1)
            start_store(b, buf1, s1)
            drain_store(a, buf0, s0)

            @pl.when(k < _NG // 2 - 1)
            def _():
                fire_gathers(b + 1, buf0, g0)

            return carry

        lax.fori_loop(0, _NG // 2, body_k, 0)
        drain_store(_NG - 1, buf1, s1)

    return gather_kernel


_gather = _build()


def kernel(inputs, W):
    idx = inputs.reshape(-1).astype(jnp.int32)
    return _gather(W, idx)
